# Initial kernel scaffold; baseline (speedup 1.0000x reference)
import jax
import jax.numpy as jnp
from jax.experimental import pallas as pl
from jax.experimental.pallas import tpu as pltpu

B, S, DIM, HID, E = 2, 2048, 1024, 4096, 8
N = B * S
EP = 128
EPS = 1e-6

TN1 = 256
TN2 = 512
HT = 1024


def _gate_kernel(x_ref, wg_ref, bg_ref, fg_ref):
    l = jnp.dot(x_ref[...], wg_ref[...], preferred_element_type=jnp.float32,
                precision=jax.lax.Precision.HIGHEST) + bg_ref[...]
    m = jnp.max(l, axis=1, keepdims=True)
    ex = jnp.exp(l - m)
    s = ex / jnp.sum(ex, axis=1, keepdims=True)
    lane = jax.lax.broadcasted_iota(jnp.int32, s.shape, 1)
    m1 = jnp.max(s, axis=1, keepdims=True)
    i1 = jnp.min(jnp.where(s == m1, lane, EP), axis=1, keepdims=True)
    oh1 = lane == i1
    s2 = jnp.where(oh1, -1.0, s)
    m2 = jnp.max(s2, axis=1, keepdims=True)
    i2 = jnp.min(jnp.where(s2 == m2, lane, EP), axis=1, keepdims=True)
    oh2 = lane == i2
    denom = m1 + m2 + EPS
    fg_ref[...] = jnp.where(oh1, m1 / denom, 0.0) + jnp.where(oh2, m2 / denom, 0.0)


def _dense_kernel(x_ref, fg_ref, w1_ref, b1_ref, w2_ref, b2_ref, out_ref):
    e = pl.program_id(1)
    hk = pl.program_id(2)
    lane = jax.lax.broadcasted_iota(jnp.int32, (TN2, EP), 1)
    w = jnp.sum(jnp.where(lane == e, fg_ref[...], 0.0), axis=1, keepdims=True)
    xb = x_ref[...].astype(jnp.bfloat16)
    h = jnp.dot(xb, w1_ref[0], preferred_element_type=jnp.float32) + b1_ref[0]
    h = jax.nn.gelu(h, approximate=False)
    part = jnp.dot(h.astype(jnp.bfloat16), w2_ref[0],
                   preferred_element_type=jnp.float32)
    contrib = w * part
    contrib = contrib + jnp.where(hk == 0, 1.0, 0.0) * (w * b2_ref[0])
    is_first = jnp.logical_and(e == 0, hk == 0)

    @pl.when(is_first)
    def _init():
        out_ref[...] = contrib

    @pl.when(jnp.logical_not(is_first))
    def _acc():
        out_ref[...] += contrib


def _gate(x2, Wgp, bgp, interpret=False):
    return pl.pallas_call(
        _gate_kernel,
        grid=(N // TN1,),
        in_specs=[
            pl.BlockSpec((TN1, DIM), lambda t: (t, 0)),
            pl.BlockSpec((DIM, EP), lambda t: (0, 0)),
            pl.BlockSpec((1, EP), lambda t: (0, 0)),
        ],
        out_specs=pl.BlockSpec((TN1, EP), lambda t: (t, 0)),
        out_shape=jax.ShapeDtypeStruct((N, EP), jnp.float32),
        interpret=interpret,
    )(x2, Wgp, bgp)


def _dense(x2, fg, W1b, b1r, W2b, b2r, interpret=False):
    return pl.pallas_call(
        _dense_kernel,
        grid=(N // TN2, E, HID // HT),
        in_specs=[
            pl.BlockSpec((TN2, DIM), lambda t, e, h: (t, 0)),
            pl.BlockSpec((TN2, EP), lambda t, e, h: (t, 0)),
            pl.BlockSpec((1, DIM, HT), lambda t, e, h: (e, 0, h)),
            pl.BlockSpec((1, 1, HT), lambda t, e, h: (e, 0, h)),
            pl.BlockSpec((1, HT, DIM), lambda t, e, h: (e, h, 0)),
            pl.BlockSpec((1, 1, DIM), lambda t, e, h: (e, 0, 0)),
        ],
        out_specs=pl.BlockSpec((TN2, DIM), lambda t, e, h: (t, 0)),
        out_shape=jax.ShapeDtypeStruct((N, DIM), jnp.float32),
        compiler_params=pltpu.CompilerParams(
            dimension_semantics=("parallel", "arbitrary", "arbitrary")),
        interpret=interpret,
    )(x2, fg, W1b, b1r, W2b, b2r)


def kernel(x, Wg, bg, W1, b1, W2, b2, interpret=False):
    x2 = x.reshape(N, DIM)
    Wgp = jnp.zeros((DIM, EP), jnp.float32).at[:, :E].set(Wg)
    bgp = jnp.full((1, EP), -1e30, jnp.float32).at[0, :E].set(bg)
    fg = _gate(x2, Wgp, bgp, interpret=interpret)
    W1b = W1.astype(jnp.bfloat16)
    W2b = W2.astype(jnp.bfloat16)
    b1r = b1.reshape(E, 1, HID)
    b2r = b2.reshape(E, 1, DIM)
    out = _dense(x2, fg, W1b, b1r, W2b, b2r, interpret=interpret)
    return out.reshape(B, S, DIM)


# speedup vs baseline: 3.1382x; 3.1382x over previous
import jax
import jax.numpy as jnp
from jax.experimental import pallas as pl
from jax.experimental.pallas import tpu as pltpu

B, S, DIM, HID, E = 2, 2048, 1024, 4096, 8
N = B * S
EP = 128
EPS = 1e-6

TN1 = 256
TN2 = 512
HT = 1024


def _gate_kernel(x_ref, wg_ref, bg_ref, fg_ref):
    l = jnp.dot(x_ref[...].astype(jnp.bfloat16), wg_ref[...].astype(jnp.bfloat16),
                preferred_element_type=jnp.float32) + bg_ref[...]
    m = jnp.max(l, axis=1, keepdims=True)
    ex = jnp.exp(l - m)
    s = ex / jnp.sum(ex, axis=1, keepdims=True)
    lane = jax.lax.broadcasted_iota(jnp.int32, s.shape, 1)
    m1 = jnp.max(s, axis=1, keepdims=True)
    i1 = jnp.min(jnp.where(s == m1, lane, EP), axis=1, keepdims=True)
    oh1 = lane == i1
    s2 = jnp.where(oh1, -1.0, s)
    m2 = jnp.max(s2, axis=1, keepdims=True)
    i2 = jnp.min(jnp.where(s2 == m2, lane, EP), axis=1, keepdims=True)
    oh2 = lane == i2
    denom = m1 + m2 + EPS
    fg_ref[...] = jnp.where(oh1, m1 / denom, 0.0) + jnp.where(oh2, m2 / denom, 0.0)


def _dense_kernel(x_ref, fg_ref, w1_ref, b1_ref, w2_ref, b2_ref, out_ref):
    e = pl.program_id(1)
    hk = pl.program_id(2)
    lane = jax.lax.broadcasted_iota(jnp.int32, (TN2, EP), 1)
    w = jnp.sum(jnp.where(lane == e, fg_ref[...], 0.0), axis=1, keepdims=True)
    xb = x_ref[...].astype(jnp.bfloat16)
    h = jnp.dot(xb, w1_ref[0], preferred_element_type=jnp.float32) + b1_ref[0]
    h = 0.5 * h * (1.0 + jax.lax.erf(h * 0.7071067811865476))
    part = jnp.dot(h.astype(jnp.bfloat16), w2_ref[0],
                   preferred_element_type=jnp.float32)
    contrib = w * part
    contrib = contrib + jnp.where(hk == 0, 1.0, 0.0) * (w * b2_ref[0])
    is_first = jnp.logical_and(e == 0, hk == 0)

    @pl.when(is_first)
    def _init():
        out_ref[...] = contrib

    @pl.when(jnp.logical_not(is_first))
    def _acc():
        out_ref[...] += contrib


def _gate(x2, Wgp, bgp, interpret=False):
    return pl.pallas_call(
        _gate_kernel,
        grid=(N // TN1,),
        in_specs=[
            pl.BlockSpec((TN1, DIM), lambda t: (t, 0)),
            pl.BlockSpec((DIM, EP), lambda t: (0, 0)),
            pl.BlockSpec((1, EP), lambda t: (0, 0)),
        ],
        out_specs=pl.BlockSpec((TN1, EP), lambda t: (t, 0)),
        out_shape=jax.ShapeDtypeStruct((N, EP), jnp.float32),
        interpret=interpret,
    )(x2, Wgp, bgp)


def _dense(x2, fg, W1b, b1r, W2b, b2r, interpret=False):
    return pl.pallas_call(
        _dense_kernel,
        grid=(N // TN2, E, HID // HT),
        in_specs=[
            pl.BlockSpec((TN2, DIM), lambda t, e, h: (t, 0)),
            pl.BlockSpec((TN2, EP), lambda t, e, h: (t, 0)),
            pl.BlockSpec((1, DIM, HT), lambda t, e, h: (e, 0, h)),
            pl.BlockSpec((1, 1, HT), lambda t, e, h: (e, 0, h)),
            pl.BlockSpec((1, HT, DIM), lambda t, e, h: (e, h, 0)),
            pl.BlockSpec((1, 1, DIM), lambda t, e, h: (e, 0, 0)),
        ],
        out_specs=pl.BlockSpec((TN2, DIM), lambda t, e, h: (t, 0)),
        out_shape=jax.ShapeDtypeStruct((N, DIM), jnp.float32),
        compiler_params=pltpu.CompilerParams(
            dimension_semantics=("parallel", "arbitrary", "arbitrary")),
        interpret=interpret,
    )(x2, fg, W1b, b1r, W2b, b2r)


def kernel(x, Wg, bg, W1, b1, W2, b2, interpret=False):
    x2 = x.reshape(N, DIM)
    Wgp = jnp.zeros((DIM, EP), jnp.float32).at[:, :E].set(Wg)
    bgp = jnp.full((1, EP), -1e30, jnp.float32).at[0, :E].set(bg)
    fg = _gate(x2, Wgp, bgp, interpret=interpret)
    W1b = W1.astype(jnp.bfloat16)
    W2b = W2.astype(jnp.bfloat16)
    b1r = b1.reshape(E, 1, HID)
    b2r = b2.reshape(E, 1, DIM)
    out = _dense(x2, fg, W1b, b1r, W2b, b2r, interpret=interpret)
    return out.reshape(B, S, DIM)


# speedup vs baseline: 5.0656x; 1.6142x over previous
import jax
import jax.numpy as jnp
from jax import lax
from jax.experimental import pallas as pl
from jax.experimental.pallas import tpu as pltpu
from jax.experimental.pallas import tpu_sc as plsc

B, S, DIM, HID, E = 2, 2048, 1024, 4096, 8
N = B * S
EP = 128
EPS = 1e-6

TN1 = 256
NGT = N // TN1
T = 256
NT = (2 * N) // T + E
NTP = 48
ROWS = NT * T
HK = 2
HT = HID // HK
TN2 = 512

NC, NS = 2, 16
NW = NC * NS
TOK_W = N // NW
_SQRT1_2 = 0.7071067811865476


def _gate_kernel(x_ref, wg_ref, bg_ref,
                 rows0_ref, rows1_ref, wc0_ref, wc1_ref, te_ref, tv_ref,
                 carry, offs_s, st_e0, st_e1, st_r0, st_r1, st_w0, st_w1):
    p = pl.program_id(0)
    t = pl.program_id(1)
    sub = lax.broadcasted_iota(jnp.int32, (EP, TN1), 0)

    @pl.when(p == 0)
    def _phase0():
        lt = jnp.dot(x_ref[...].astype(jnp.bfloat16),
                     wg_ref[...].astype(jnp.bfloat16),
                     preferred_element_type=jnp.float32)
        l = lt.T + bg_ref[...]
        m = jnp.max(l, axis=0, keepdims=True)
        ex = jnp.exp(l - m)
        s = ex / jnp.sum(ex, axis=0, keepdims=True)
        m1 = jnp.max(s, axis=0, keepdims=True)
        i1 = jnp.min(jnp.where(s == m1, sub, EP), axis=0, keepdims=True)
        oh1 = sub == i1
        sm = jnp.where(oh1, -1.0, s)
        m2 = jnp.max(sm, axis=0, keepdims=True)
        i2 = jnp.min(jnp.where(sm == m2, sub, EP), axis=0, keepdims=True)
        oh2 = sub == i2
        denom = m1 + m2 + EPS

        @pl.when(t == 0)
        def _():
            carry[...] = jnp.zeros_like(carry)

        oh1f = oh1.astype(jnp.float32)
        oh2f = oh2.astype(jnp.float32)
        ohsum = oh1f + oh2f
        rowi = lax.broadcasted_iota(jnp.int32, (TN1, TN1), 0)
        coli = lax.broadcasted_iota(jnp.int32, (TN1, TN1), 1)
        ut = (rowi < coli).astype(jnp.float32)
        excl = jnp.dot(ohsum, ut, preferred_element_type=jnp.float32)
        cbase = carry[...] + excl
        st_e0[pl.ds(t, 1), :] = i1.astype(jnp.float32)
        st_e1[pl.ds(t, 1), :] = i2.astype(jnp.float32)
        st_r0[pl.ds(t, 1), :] = jnp.sum(oh1f * cbase, axis=0, keepdims=True)
        st_r1[pl.ds(t, 1), :] = jnp.sum(oh2f * (cbase + oh1f), axis=0,
                                        keepdims=True)
        st_w0[pl.ds(t, 1), :] = m1 / denom
        st_w1[pl.ds(t, 1), :] = m2 / denom
        carry[...] = carry[...] + jnp.sum(ohsum, axis=1, keepdims=True)

    @pl.when(jnp.logical_and(p == 1, t == 0))
    def _offsets():
        padded = jnp.floor((carry[...] + (T - 1)) * (1.0 / T)) * T
        rowe = lax.broadcasted_iota(jnp.int32, (EP, EP), 0)
        cole = lax.broadcasted_iota(jnp.int32, (EP, EP), 1)
        ltm = (cole < rowe).astype(jnp.float32)
        offs_s[...] = jnp.dot(ltm, padded, preferred_element_type=jnp.float32,
                              precision=jax.lax.Precision.HIGHEST)

    @pl.when(p == 1)
    def _phase1():
        e0r = st_e0[pl.ds(t, 1), :]
        e1r = st_e1[pl.ds(t, 1), :]
        oh0 = (sub == e0r.astype(jnp.int32)).astype(jnp.float32)
        oh1 = (sub == e1r.astype(jnp.int32)).astype(jnp.float32)
        off0 = jnp.sum(oh0 * offs_s[...], axis=0, keepdims=True)
        off1 = jnp.sum(oh1 * offs_s[...], axis=0, keepdims=True)
        row0 = off0 + st_r0[pl.ds(t, 1), :]
        row1 = off1 + st_r1[pl.ds(t, 1), :]
        rows0_ref[...] = row0.T.astype(jnp.int32)
        rows1_ref[...] = row1.T.astype(jnp.int32)
        wc0_ref[...] = st_w0[pl.ds(t, 1), :].T
        wc1_ref[...] = st_w1[pl.ds(t, 1), :].T

        @pl.when(t == pl.num_programs(1) - 1)
        def _tilemaps():
            padded = jnp.floor((carry[...] + (T - 1)) * (1.0 / T)) * T
            endoff = offs_s[...] + padded
            subc = lax.broadcasted_iota(jnp.int32, (EP, 1), 0)
            it_col = (subc * T).astype(jnp.float32)
            cmp = (it_col >= endoff.T).astype(jnp.float32)
            acc = jnp.sum(cmp, axis=1, keepdims=True)
            te_ref[...] = jnp.minimum(acc, float(E - 1)).astype(jnp.int32)
            total = jnp.max(endoff)
            tv_ref[...] = (it_col < total).astype(jnp.int32)


def _gate(x2, Wgp, bgp, interpret=False):
    return pl.pallas_call(
        _gate_kernel,
        grid=(2, NGT),
        in_specs=[
            pl.BlockSpec((TN1, DIM), lambda p, t: (t, 0)),
            pl.BlockSpec((DIM, EP), lambda p, t: (0, 0)),
            pl.BlockSpec((EP, 1), lambda p, t: (0, 0)),
        ],
        out_specs=[
            pl.BlockSpec((TN1, 1), lambda p, t: (t, 0)),
            pl.BlockSpec((TN1, 1), lambda p, t: (t, 0)),
            pl.BlockSpec((TN1, 1), lambda p, t: (t, 0)),
            pl.BlockSpec((TN1, 1), lambda p, t: (t, 0)),
            pl.BlockSpec((EP, 1), lambda p, t: (0, 0)),
            pl.BlockSpec((EP, 1), lambda p, t: (0, 0)),
        ],
        out_shape=[
            jax.ShapeDtypeStruct((N, 1), jnp.int32),
            jax.ShapeDtypeStruct((N, 1), jnp.int32),
            jax.ShapeDtypeStruct((N, 1), jnp.float32),
            jax.ShapeDtypeStruct((N, 1), jnp.float32),
            jax.ShapeDtypeStruct((EP, 1), jnp.int32),
            jax.ShapeDtypeStruct((EP, 1), jnp.int32),
        ],
        scratch_shapes=[
            pltpu.VMEM((EP, 1), jnp.float32),
            pltpu.VMEM((EP, 1), jnp.float32),
            pltpu.VMEM((NGT, TN1), jnp.float32),
            pltpu.VMEM((NGT, TN1), jnp.float32),
            pltpu.VMEM((NGT, TN1), jnp.float32),
            pltpu.VMEM((NGT, TN1), jnp.float32),
            pltpu.VMEM((NGT, TN1), jnp.float32),
            pltpu.VMEM((NGT, TN1), jnp.float32),
        ],
        interpret=interpret,
    )(x2, Wgp, bgp)


def _scatter_body(x2_hbm, rows0_hbm, rows1_hbm, xs_hbm,
                  i0_v, i1_v, idx0_v, idx1_v, xrow_v, sem):
    c = lax.axis_index("c")
    s_ = lax.axis_index("s")
    wid = s_ * NC + c
    base = wid * TOK_W

    pltpu.sync_copy(rows0_hbm.at[pl.ds(base, TOK_W)], i0_v)
    pltpu.sync_copy(rows1_hbm.at[pl.ds(base, TOK_W)], i1_v)
    for ch in range(TOK_W // 16):
        sl = pl.ds(ch * 16, 16)
        pltpu.sync_copy(x2_hbm.at[pl.ds(base + ch * 16, 16)], xrow_v)
        idx0_v[...] = i0_v[sl]
        pltpu.async_copy(xrow_v, xs_hbm.at[idx0_v], sem).wait()
        idx1_v[...] = i1_v[sl]
        pltpu.async_copy(xrow_v, xs_hbm.at[idx1_v], sem).wait()


def _scatter(x2, rows0, rows1, interpret=False):
    del interpret
    mesh = plsc.VectorSubcoreMesh(core_axis_name="c", subcore_axis_name="s")
    return pl.kernel(
        _scatter_body,
        out_type=jax.ShapeDtypeStruct((ROWS, DIM), jnp.float32),
        mesh=mesh,
        scratch_types=[
            pltpu.VMEM((TOK_W,), jnp.int32),
            pltpu.VMEM((TOK_W,), jnp.int32),
            pltpu.VMEM((16,), jnp.int32),
            pltpu.VMEM((16,), jnp.int32),
            pltpu.VMEM((16, DIM), jnp.float32),
            pltpu.SemaphoreType.DMA,
        ],
    )(x2, rows0, rows1)


def _gemm_kernel(te_ref, tv_ref, xs_ref, w1_ref, b1_ref, w2_ref, b2_ref, y_ref):
    i = pl.program_id(0)
    hk = pl.program_id(1)

    @pl.when(tv_ref[i] == 1)
    def _():
        xb = xs_ref[...].astype(jnp.bfloat16)
        h = jnp.dot(xb, w1_ref[0], preferred_element_type=jnp.float32) + b1_ref[0]
        h = 0.5 * h * (1.0 + lax.erf(h * _SQRT1_2))
        part = jnp.dot(h.astype(jnp.bfloat16), w2_ref[0],
                       preferred_element_type=jnp.float32)

        @pl.when(hk == 0)
        def _():
            y_ref[...] = part + b2_ref[0]

        @pl.when(hk > 0)
        def _():
            y_ref[...] += part


def _gemm(te, tv, xs, W1b, b1r, W2b, b2r, interpret=False):
    grid_spec = pltpu.PrefetchScalarGridSpec(
        num_scalar_prefetch=2,
        grid=(NT, HK),
        in_specs=[
            pl.BlockSpec((T, DIM), lambda i, hk, te, tv: (i, 0)),
            pl.BlockSpec((1, DIM, HT), lambda i, hk, te, tv: (te[i], 0, hk)),
            pl.BlockSpec((1, 1, HT), lambda i, hk, te, tv: (te[i], 0, hk)),
            pl.BlockSpec((1, HT, DIM), lambda i, hk, te, tv: (te[i], hk, 0)),
            pl.BlockSpec((1, 1, DIM), lambda i, hk, te, tv: (te[i], 0, 0)),
        ],
        out_specs=pl.BlockSpec((T, DIM), lambda i, hk, te, tv: (i, 0)),
    )
    return pl.pallas_call(
        _gemm_kernel,
        grid_spec=grid_spec,
        out_shape=jax.ShapeDtypeStruct((ROWS, DIM), jnp.float32),
        compiler_params=pltpu.CompilerParams(
            dimension_semantics=("arbitrary", "arbitrary")),
        interpret=interpret,
    )(te, tv, xs, W1b, b1r, W2b, b2r)


def _gather_body(y_hbm, rows0_hbm, rows1_hbm, g0_hbm, g1_hbm,
                 i0_v, i1_v, idx0_v, idx1_v, y0_v, y1_v, sem):
    c = lax.axis_index("c")
    s_ = lax.axis_index("s")
    wid = s_ * NC + c
    base = wid * TOK_W

    pltpu.sync_copy(rows0_hbm.at[pl.ds(base, TOK_W)], i0_v)
    pltpu.sync_copy(rows1_hbm.at[pl.ds(base, TOK_W)], i1_v)
    for ch in range(TOK_W // 16):
        sl = pl.ds(ch * 16, 16)
        idx0_v[...] = i0_v[sl]
        idx1_v[...] = i1_v[sl]
        pltpu.async_copy(y_hbm.at[idx0_v], y0_v, sem).wait()
        pltpu.async_copy(y_hbm.at[idx1_v], y1_v, sem).wait()
        pltpu.sync_copy(y0_v, g0_hbm.at[pl.ds(base + ch * 16, 16)])
        pltpu.sync_copy(y1_v, g1_hbm.at[pl.ds(base + ch * 16, 16)])


def _gather(y, rows0, rows1, interpret=False):
    del interpret
    mesh = plsc.VectorSubcoreMesh(core_axis_name="c", subcore_axis_name="s")
    return pl.kernel(
        _gather_body,
        out_type=[
            jax.ShapeDtypeStruct((N, DIM), jnp.float32),
            jax.ShapeDtypeStruct((N, DIM), jnp.float32),
        ],
        mesh=mesh,
        scratch_types=[
            pltpu.VMEM((TOK_W,), jnp.int32),
            pltpu.VMEM((TOK_W,), jnp.int32),
            pltpu.VMEM((16,), jnp.int32),
            pltpu.VMEM((16,), jnp.int32),
            pltpu.VMEM((16, DIM), jnp.float32),
            pltpu.VMEM((16, DIM), jnp.float32),
            pltpu.SemaphoreType.DMA,
        ],
    )(y, rows0, rows1)


def _mix_kernel(g0_ref, g1_ref, wc0_ref, wc1_ref, out_ref):
    out_ref[...] = wc0_ref[...] * g0_ref[...] + wc1_ref[...] * g1_ref[...]


def _mix(g0, g1, wc0, wc1, interpret=False):
    return pl.pallas_call(
        _mix_kernel,
        grid=(N // TN2,),
        in_specs=[
            pl.BlockSpec((TN2, DIM), lambda t: (t, 0)),
            pl.BlockSpec((TN2, DIM), lambda t: (t, 0)),
            pl.BlockSpec((TN2, 1), lambda t: (t, 0)),
            pl.BlockSpec((TN2, 1), lambda t: (t, 0)),
        ],
        out_specs=pl.BlockSpec((TN2, DIM), lambda t: (t, 0)),
        out_shape=jax.ShapeDtypeStruct((N, DIM), jnp.float32),
        interpret=interpret,
    )(g0, g1, wc0, wc1)


def kernel(x, Wg, bg, W1, b1, W2, b2, interpret=False):
    x2 = x.reshape(N, DIM)
    Wgp = jnp.zeros((DIM, EP), jnp.float32).at[:, :E].set(Wg)
    bgp = jnp.full((EP, 1), -1e30, jnp.float32).at[:E, 0].set(bg)
    rows0, rows1, wc0, wc1, te, tv = _gate(x2, Wgp, bgp, interpret=interpret)
    r0f = rows0.reshape(N)
    r1f = rows1.reshape(N)
    xs = _scatter(x2, r0f, r1f, interpret=interpret)
    W1b = W1.astype(jnp.bfloat16)
    W2b = W2.astype(jnp.bfloat16)
    b1r = b1.reshape(E, 1, HID)
    b2r = b2.reshape(E, 1, DIM)
    y = _gemm(te.reshape(EP)[:NTP], tv.reshape(EP)[:NTP], xs, W1b, b1r, W2b,
              b2r, interpret=interpret)
    g0, g1 = _gather(y, r0f, r1f, interpret=interpret)
    out2 = _mix(g0, g1, wc0, wc1, interpret=interpret)
    return out2.reshape(B, S, DIM)


# speedup vs baseline: 5.8065x; 1.1463x over previous
import jax
import jax.numpy as jnp
from jax import lax
from jax.experimental import pallas as pl
from jax.experimental.pallas import tpu as pltpu
from jax.experimental.pallas import tpu_sc as plsc

B, S, DIM, HID, E = 2, 2048, 1024, 4096, 8
N = B * S
EP = 128
EPS = 1e-6

TN1 = 256
NGT = N // TN1
T = 256
NT = (2 * N) // T + E
NTP = 48
ROWS = NT * T
HK = 1
HT = HID // HK
TN2 = 512

NC, NS = 2, 16
NW = NC * NS
TOK_W = N // NW
_SQRT1_2 = 0.7071067811865476


def _gate_kernel(x_ref, wg_ref, bg_ref,
                 rows0_ref, rows1_ref, wc0_ref, wc1_ref, te_ref, tv_ref,
                 carry, offs_s, st_e0, st_e1, st_r0, st_r1, st_w0, st_w1):
    p = pl.program_id(0)
    t = pl.program_id(1)
    sub = lax.broadcasted_iota(jnp.int32, (EP, TN1), 0)

    @pl.when(p == 0)
    def _phase0():
        lt = jnp.dot(x_ref[...].astype(jnp.bfloat16),
                     wg_ref[...].astype(jnp.bfloat16),
                     preferred_element_type=jnp.float32)
        l = lt.T + bg_ref[...]
        m = jnp.max(l, axis=0, keepdims=True)
        ex = jnp.exp(l - m)
        s = ex / jnp.sum(ex, axis=0, keepdims=True)
        m1 = jnp.max(s, axis=0, keepdims=True)
        i1 = jnp.min(jnp.where(s == m1, sub, EP), axis=0, keepdims=True)
        oh1 = sub == i1
        sm = jnp.where(oh1, -1.0, s)
        m2 = jnp.max(sm, axis=0, keepdims=True)
        i2 = jnp.min(jnp.where(sm == m2, sub, EP), axis=0, keepdims=True)
        oh2 = sub == i2
        denom = m1 + m2 + EPS

        @pl.when(t == 0)
        def _():
            carry[...] = jnp.zeros_like(carry)

        oh1f = oh1.astype(jnp.float32)
        oh2f = oh2.astype(jnp.float32)
        ohsum = oh1f + oh2f
        rowi = lax.broadcasted_iota(jnp.int32, (TN1, TN1), 0)
        coli = lax.broadcasted_iota(jnp.int32, (TN1, TN1), 1)
        ut = (rowi < coli).astype(jnp.float32)
        excl = jnp.dot(ohsum, ut, preferred_element_type=jnp.float32)
        cbase = carry[...] + excl
        st_e0[pl.ds(t, 1), :] = i1.astype(jnp.float32)
        st_e1[pl.ds(t, 1), :] = i2.astype(jnp.float32)
        st_r0[pl.ds(t, 1), :] = jnp.sum(oh1f * cbase, axis=0, keepdims=True)
        st_r1[pl.ds(t, 1), :] = jnp.sum(oh2f * (cbase + oh1f), axis=0,
                                        keepdims=True)
        st_w0[pl.ds(t, 1), :] = m1 / denom
        st_w1[pl.ds(t, 1), :] = m2 / denom
        carry[...] = carry[...] + jnp.sum(ohsum, axis=1, keepdims=True)

    @pl.when(jnp.logical_and(p == 1, t == 0))
    def _offsets():
        padded = jnp.floor((carry[...] + (T - 1)) * (1.0 / T)) * T
        rowe = lax.broadcasted_iota(jnp.int32, (EP, EP), 0)
        cole = lax.broadcasted_iota(jnp.int32, (EP, EP), 1)
        ltm = (cole < rowe).astype(jnp.float32)
        offs_s[...] = jnp.dot(ltm, padded, preferred_element_type=jnp.float32,
                              precision=jax.lax.Precision.HIGHEST)

    @pl.when(p == 1)
    def _phase1():
        e0r = st_e0[pl.ds(t, 1), :]
        e1r = st_e1[pl.ds(t, 1), :]
        oh0 = (sub == e0r.astype(jnp.int32)).astype(jnp.float32)
        oh1 = (sub == e1r.astype(jnp.int32)).astype(jnp.float32)
        off0 = jnp.sum(oh0 * offs_s[...], axis=0, keepdims=True)
        off1 = jnp.sum(oh1 * offs_s[...], axis=0, keepdims=True)
        row0 = off0 + st_r0[pl.ds(t, 1), :]
        row1 = off1 + st_r1[pl.ds(t, 1), :]
        rows0_ref[...] = row0.T.astype(jnp.int32)
        rows1_ref[...] = row1.T.astype(jnp.int32)
        wc0_ref[...] = st_w0[pl.ds(t, 1), :].T
        wc1_ref[...] = st_w1[pl.ds(t, 1), :].T

        @pl.when(t == pl.num_programs(1) - 1)
        def _tilemaps():
            padded = jnp.floor((carry[...] + (T - 1)) * (1.0 / T)) * T
            endoff = offs_s[...] + padded
            subc = lax.broadcasted_iota(jnp.int32, (EP, 1), 0)
            it_col = (subc * T).astype(jnp.float32)
            cmp = (it_col >= endoff.T).astype(jnp.float32)
            acc = jnp.sum(cmp, axis=1, keepdims=True)
            te_ref[...] = jnp.minimum(acc, float(E - 1)).astype(jnp.int32)
            total = jnp.max(endoff)
            tv_ref[...] = (it_col < total).astype(jnp.int32)


def _gate(x2, Wgp, bgp, interpret=False):
    return pl.pallas_call(
        _gate_kernel,
        grid=(2, NGT),
        in_specs=[
            pl.BlockSpec((TN1, DIM), lambda p, t: (t, 0)),
            pl.BlockSpec((DIM, EP), lambda p, t: (0, 0)),
            pl.BlockSpec((EP, 1), lambda p, t: (0, 0)),
        ],
        out_specs=[
            pl.BlockSpec((TN1, 1), lambda p, t: (t, 0)),
            pl.BlockSpec((TN1, 1), lambda p, t: (t, 0)),
            pl.BlockSpec((TN1, 1), lambda p, t: (t, 0)),
            pl.BlockSpec((TN1, 1), lambda p, t: (t, 0)),
            pl.BlockSpec((EP, 1), lambda p, t: (0, 0)),
            pl.BlockSpec((EP, 1), lambda p, t: (0, 0)),
        ],
        out_shape=[
            jax.ShapeDtypeStruct((N, 1), jnp.int32),
            jax.ShapeDtypeStruct((N, 1), jnp.int32),
            jax.ShapeDtypeStruct((N, 1), jnp.float32),
            jax.ShapeDtypeStruct((N, 1), jnp.float32),
            jax.ShapeDtypeStruct((EP, 1), jnp.int32),
            jax.ShapeDtypeStruct((EP, 1), jnp.int32),
        ],
        scratch_shapes=[
            pltpu.VMEM((EP, 1), jnp.float32),
            pltpu.VMEM((EP, 1), jnp.float32),
            pltpu.VMEM((NGT, TN1), jnp.float32),
            pltpu.VMEM((NGT, TN1), jnp.float32),
            pltpu.VMEM((NGT, TN1), jnp.float32),
            pltpu.VMEM((NGT, TN1), jnp.float32),
            pltpu.VMEM((NGT, TN1), jnp.float32),
            pltpu.VMEM((NGT, TN1), jnp.float32),
        ],
        interpret=interpret,
    )(x2, Wgp, bgp)


def _scatter_body(x2_hbm, rows0_hbm, rows1_hbm, xs_hbm,
                  i0_v, i1_v, idx0_v, idx1_v, xrow_v, sem):
    c = lax.axis_index("c")
    s_ = lax.axis_index("s")
    wid = s_ * NC + c
    base = wid * TOK_W

    pltpu.sync_copy(rows0_hbm.at[pl.ds(base, TOK_W)], i0_v)
    pltpu.sync_copy(rows1_hbm.at[pl.ds(base, TOK_W)], i1_v)
    for ch in range(TOK_W // 16):
        sl = pl.ds(ch * 16, 16)
        pltpu.sync_copy(x2_hbm.at[pl.ds(base + ch * 16, 16)], xrow_v)
        idx0_v[...] = i0_v[sl]
        pltpu.async_copy(xrow_v, xs_hbm.at[idx0_v], sem).wait()
        idx1_v[...] = i1_v[sl]
        pltpu.async_copy(xrow_v, xs_hbm.at[idx1_v], sem).wait()


def _scatter(x2, rows0, rows1, interpret=False):
    del interpret
    mesh = plsc.VectorSubcoreMesh(core_axis_name="c", subcore_axis_name="s")
    return pl.kernel(
        _scatter_body,
        out_type=jax.ShapeDtypeStruct((ROWS, DIM), jnp.float32),
        mesh=mesh,
        scratch_types=[
            pltpu.VMEM((TOK_W,), jnp.int32),
            pltpu.VMEM((TOK_W,), jnp.int32),
            pltpu.VMEM((16,), jnp.int32),
            pltpu.VMEM((16,), jnp.int32),
            pltpu.VMEM((16, DIM), jnp.float32),
            pltpu.SemaphoreType.DMA,
        ],
    )(x2, rows0, rows1)


def _gemm_kernel(te_ref, tv_ref, xs_ref, w1_ref, b1_ref, w2_ref, b2_ref, y_ref):
    i = pl.program_id(0)

    @pl.when(tv_ref[i] == 1)
    def _():
        xb = xs_ref[...].astype(jnp.bfloat16)
        h = jnp.dot(xb, w1_ref[0], preferred_element_type=jnp.float32) + b1_ref[0]
        h = 0.5 * h * (1.0 + lax.erf(h * _SQRT1_2))
        y_ref[...] = jnp.dot(h.astype(jnp.bfloat16), w2_ref[0],
                             preferred_element_type=jnp.float32) + b2_ref[0]


def _gemm(te, tv, xs, W1b, b1r, W2b, b2r, interpret=False):
    grid_spec = pltpu.PrefetchScalarGridSpec(
        num_scalar_prefetch=2,
        grid=(NT,),
        in_specs=[
            pl.BlockSpec((T, DIM), lambda i, te, tv: (i, 0)),
            pl.BlockSpec((1, DIM, HT), lambda i, te, tv: (te[i], 0, 0)),
            pl.BlockSpec((1, 1, HT), lambda i, te, tv: (te[i], 0, 0)),
            pl.BlockSpec((1, HT, DIM), lambda i, te, tv: (te[i], 0, 0)),
            pl.BlockSpec((1, 1, DIM), lambda i, te, tv: (te[i], 0, 0)),
        ],
        out_specs=pl.BlockSpec((T, DIM), lambda i, te, tv: (i, 0)),
    )
    return pl.pallas_call(
        _gemm_kernel,
        grid_spec=grid_spec,
        out_shape=jax.ShapeDtypeStruct((ROWS, DIM), jnp.float32),
        compiler_params=pltpu.CompilerParams(
            dimension_semantics=("arbitrary",)),
        interpret=interpret,
    )(te, tv, xs, W1b, b1r, W2b, b2r)


def _gather_body(y_hbm, rows0_hbm, rows1_hbm, g0_hbm, g1_hbm,
                 i0_v, i1_v, idx0_v, idx1_v, y0_v, y1_v, sem):
    c = lax.axis_index("c")
    s_ = lax.axis_index("s")
    wid = s_ * NC + c
    base = wid * TOK_W

    pltpu.sync_copy(rows0_hbm.at[pl.ds(base, TOK_W)], i0_v)
    pltpu.sync_copy(rows1_hbm.at[pl.ds(base, TOK_W)], i1_v)
    for ch in range(TOK_W // 16):
        sl = pl.ds(ch * 16, 16)
        idx0_v[...] = i0_v[sl]
        idx1_v[...] = i1_v[sl]
        pltpu.async_copy(y_hbm.at[idx0_v], y0_v, sem).wait()
        pltpu.async_copy(y_hbm.at[idx1_v], y1_v, sem).wait()
        pltpu.sync_copy(y0_v, g0_hbm.at[pl.ds(base + ch * 16, 16)])
        pltpu.sync_copy(y1_v, g1_hbm.at[pl.ds(base + ch * 16, 16)])


def _gather(y, rows0, rows1, interpret=False):
    del interpret
    mesh = plsc.VectorSubcoreMesh(core_axis_name="c", subcore_axis_name="s")
    return pl.kernel(
        _gather_body,
        out_type=[
            jax.ShapeDtypeStruct((N, DIM), jnp.float32),
            jax.ShapeDtypeStruct((N, DIM), jnp.float32),
        ],
        mesh=mesh,
        scratch_types=[
            pltpu.VMEM((TOK_W,), jnp.int32),
            pltpu.VMEM((TOK_W,), jnp.int32),
            pltpu.VMEM((16,), jnp.int32),
            pltpu.VMEM((16,), jnp.int32),
            pltpu.VMEM((16, DIM), jnp.float32),
            pltpu.VMEM((16, DIM), jnp.float32),
            pltpu.SemaphoreType.DMA,
        ],
    )(y, rows0, rows1)


def _mix_kernel(g0_ref, g1_ref, wc0_ref, wc1_ref, out_ref):
    out_ref[...] = wc0_ref[...] * g0_ref[...] + wc1_ref[...] * g1_ref[...]


def _mix(g0, g1, wc0, wc1, interpret=False):
    return pl.pallas_call(
        _mix_kernel,
        grid=(N // TN2,),
        in_specs=[
            pl.BlockSpec((TN2, DIM), lambda t: (t, 0)),
            pl.BlockSpec((TN2, DIM), lambda t: (t, 0)),
            pl.BlockSpec((TN2, 1), lambda t: (t, 0)),
            pl.BlockSpec((TN2, 1), lambda t: (t, 0)),
        ],
        out_specs=pl.BlockSpec((TN2, DIM), lambda t: (t, 0)),
        out_shape=jax.ShapeDtypeStruct((N, DIM), jnp.float32),
        interpret=interpret,
    )(g0, g1, wc0, wc1)


def kernel(x, Wg, bg, W1, b1, W2, b2, interpret=False):
    x2 = x.reshape(N, DIM)
    Wgp = jnp.zeros((DIM, EP), jnp.float32).at[:, :E].set(Wg)
    bgp = jnp.full((EP, 1), -1e30, jnp.float32).at[:E, 0].set(bg)
    rows0, rows1, wc0, wc1, te, tv = _gate(x2, Wgp, bgp, interpret=interpret)
    r0f = rows0.reshape(N)
    r1f = rows1.reshape(N)
    xs = _scatter(x2, r0f, r1f, interpret=interpret)
    W1b = W1.astype(jnp.bfloat16)
    W2b = W2.astype(jnp.bfloat16)
    b1r = b1.reshape(E, 1, HID)
    b2r = b2.reshape(E, 1, DIM)
    y = _gemm(te.reshape(EP)[:NTP], tv.reshape(EP)[:NTP], xs, W1b, b1r, W2b,
              b2r, interpret=interpret)
    g0, g1 = _gather(y, r0f, r1f, interpret=interpret)
    out2 = _mix(g0, g1, wc0, wc1, interpret=interpret)
    return out2.reshape(B, S, DIM)


# speedup vs baseline: 5.9248x; 1.0204x over previous
import jax
import jax.numpy as jnp
from jax import lax
from jax.experimental import pallas as pl
from jax.experimental.pallas import tpu as pltpu
from jax.experimental.pallas import tpu_sc as plsc

B, S, DIM, HID, E = 2, 2048, 1024, 4096, 8
N = B * S
EP = 128
EPS = 1e-6

TN1 = 256
NGT = N // TN1
T = 256
NT = (2 * N) // T + E
NTP = 48
ROWS = NT * T
HK = 1
HT = HID // HK
TN2 = 512

NC, NS = 2, 16
NW = NC * NS
TOK_W = N // NW
CH = 32
_SQRT1_2 = 0.7071067811865476


def _gate_kernel(x_ref, wg_ref, bg_ref,
                 rows0_ref, rows1_ref, wc0_ref, wc1_ref, te_ref, tv_ref,
                 carry, offs_s, st_e0, st_e1, st_r0, st_r1, st_w0, st_w1):
    p = pl.program_id(0)
    t = pl.program_id(1)
    sub = lax.broadcasted_iota(jnp.int32, (EP, TN1), 0)

    @pl.when(p == 0)
    def _phase0():
        lt = jnp.dot(x_ref[...].astype(jnp.bfloat16),
                     wg_ref[...].astype(jnp.bfloat16),
                     preferred_element_type=jnp.float32)
        l = lt.T + bg_ref[...]
        m = jnp.max(l, axis=0, keepdims=True)
        ex = jnp.exp(l - m)
        s = ex / jnp.sum(ex, axis=0, keepdims=True)
        m1 = jnp.max(s, axis=0, keepdims=True)
        i1 = jnp.min(jnp.where(s == m1, sub, EP), axis=0, keepdims=True)
        oh1 = sub == i1
        sm = jnp.where(oh1, -1.0, s)
        m2 = jnp.max(sm, axis=0, keepdims=True)
        i2 = jnp.min(jnp.where(sm == m2, sub, EP), axis=0, keepdims=True)
        oh2 = sub == i2
        denom = m1 + m2 + EPS

        @pl.when(t == 0)
        def _():
            carry[...] = jnp.zeros_like(carry)

        oh1f = oh1.astype(jnp.float32)
        oh2f = oh2.astype(jnp.float32)
        ohsum = oh1f + oh2f
        rowi = lax.broadcasted_iota(jnp.int32, (TN1, TN1), 0)
        coli = lax.broadcasted_iota(jnp.int32, (TN1, TN1), 1)
        ut = (rowi < coli).astype(jnp.float32)
        excl = jnp.dot(ohsum, ut, preferred_element_type=jnp.float32)
        cbase = carry[...] + excl
        st_e0[pl.ds(t, 1), :] = i1.astype(jnp.float32)
        st_e1[pl.ds(t, 1), :] = i2.astype(jnp.float32)
        st_r0[pl.ds(t, 1), :] = jnp.sum(oh1f * cbase, axis=0, keepdims=True)
        st_r1[pl.ds(t, 1), :] = jnp.sum(oh2f * (cbase + oh1f), axis=0,
                                        keepdims=True)
        st_w0[pl.ds(t, 1), :] = m1 / denom
        st_w1[pl.ds(t, 1), :] = m2 / denom
        carry[...] = carry[...] + jnp.sum(ohsum, axis=1, keepdims=True)

    @pl.when(jnp.logical_and(p == 1, t == 0))
    def _offsets():
        padded = jnp.floor((carry[...] + (T - 1)) * (1.0 / T)) * T
        rowe = lax.broadcasted_iota(jnp.int32, (EP, EP), 0)
        cole = lax.broadcasted_iota(jnp.int32, (EP, EP), 1)
        ltm = (cole < rowe).astype(jnp.float32)
        offs_s[...] = jnp.dot(ltm, padded, preferred_element_type=jnp.float32,
                              precision=jax.lax.Precision.HIGHEST)

    @pl.when(p == 1)
    def _phase1():
        e0r = st_e0[pl.ds(t, 1), :]
        e1r = st_e1[pl.ds(t, 1), :]
        oh0 = (sub == e0r.astype(jnp.int32)).astype(jnp.float32)
        oh1 = (sub == e1r.astype(jnp.int32)).astype(jnp.float32)
        off0 = jnp.sum(oh0 * offs_s[...], axis=0, keepdims=True)
        off1 = jnp.sum(oh1 * offs_s[...], axis=0, keepdims=True)
        row0 = off0 + st_r0[pl.ds(t, 1), :]
        row1 = off1 + st_r1[pl.ds(t, 1), :]
        rows0_ref[...] = row0.T.astype(jnp.int32)
        rows1_ref[...] = row1.T.astype(jnp.int32)
        wc0_ref[...] = st_w0[pl.ds(t, 1), :].T
        wc1_ref[...] = st_w1[pl.ds(t, 1), :].T

        @pl.when(t == pl.num_programs(1) - 1)
        def _tilemaps():
            padded = jnp.floor((carry[...] + (T - 1)) * (1.0 / T)) * T
            endoff = offs_s[...] + padded
            subc = lax.broadcasted_iota(jnp.int32, (EP, 1), 0)
            it_col = (subc * T).astype(jnp.float32)
            cmp = (it_col >= endoff.T).astype(jnp.float32)
            acc = jnp.sum(cmp, axis=1, keepdims=True)
            te_ref[...] = jnp.minimum(acc, float(E - 1)).astype(jnp.int32)
            total = jnp.max(endoff)
            tv_ref[...] = (it_col < total).astype(jnp.int32)


def _gate(x2, Wgp, bgp, interpret=False):
    return pl.pallas_call(
        _gate_kernel,
        grid=(2, NGT),
        in_specs=[
            pl.BlockSpec((TN1, DIM), lambda p, t: (t, 0)),
            pl.BlockSpec((DIM, EP), lambda p, t: (0, 0)),
            pl.BlockSpec((EP, 1), lambda p, t: (0, 0)),
        ],
        out_specs=[
            pl.BlockSpec((TN1, 1), lambda p, t: (t, 0)),
            pl.BlockSpec((TN1, 1), lambda p, t: (t, 0)),
            pl.BlockSpec((TN1, 1), lambda p, t: (t, 0)),
            pl.BlockSpec((TN1, 1), lambda p, t: (t, 0)),
            pl.BlockSpec((EP, 1), lambda p, t: (0, 0)),
            pl.BlockSpec((EP, 1), lambda p, t: (0, 0)),
        ],
        out_shape=[
            jax.ShapeDtypeStruct((N, 1), jnp.int32),
            jax.ShapeDtypeStruct((N, 1), jnp.int32),
            jax.ShapeDtypeStruct((N, 1), jnp.float32),
            jax.ShapeDtypeStruct((N, 1), jnp.float32),
            jax.ShapeDtypeStruct((EP, 1), jnp.int32),
            jax.ShapeDtypeStruct((EP, 1), jnp.int32),
        ],
        scratch_shapes=[
            pltpu.VMEM((EP, 1), jnp.float32),
            pltpu.VMEM((EP, 1), jnp.float32),
            pltpu.VMEM((NGT, TN1), jnp.float32),
            pltpu.VMEM((NGT, TN1), jnp.float32),
            pltpu.VMEM((NGT, TN1), jnp.float32),
            pltpu.VMEM((NGT, TN1), jnp.float32),
            pltpu.VMEM((NGT, TN1), jnp.float32),
            pltpu.VMEM((NGT, TN1), jnp.float32),
        ],
        interpret=interpret,
    )(x2, Wgp, bgp)


def _scatter_body(x2_hbm, rows0_hbm, rows1_hbm, xs_hbm,
                  i0_v, i1_v, idx0_v, idx1_v, xrow_v, sem):
    c = lax.axis_index("c")
    s_ = lax.axis_index("s")
    wid = s_ * NC + c
    base = wid * TOK_W

    pltpu.sync_copy(rows0_hbm.at[pl.ds(base, TOK_W)], i0_v)
    pltpu.sync_copy(rows1_hbm.at[pl.ds(base, TOK_W)], i1_v)
    for ch in range(TOK_W // CH):
        for k in range(CH // 16):
            idx0_v[pl.ds(k * 16, 16)] = i0_v[pl.ds(ch * CH + k * 16, 16)]
            idx1_v[pl.ds(k * 16, 16)] = i1_v[pl.ds(ch * CH + k * 16, 16)]
        pltpu.sync_copy(x2_hbm.at[pl.ds(base + ch * CH, CH)], xrow_v)
        cp0 = pltpu.async_copy(xrow_v, xs_hbm.at[idx0_v], sem)
        cp1 = pltpu.async_copy(xrow_v, xs_hbm.at[idx1_v], sem)
        cp0.wait()
        cp1.wait()


def _scatter(x2, rows0, rows1, interpret=False):
    del interpret
    mesh = plsc.VectorSubcoreMesh(core_axis_name="c", subcore_axis_name="s")
    return pl.kernel(
        _scatter_body,
        out_type=jax.ShapeDtypeStruct((ROWS, DIM), jnp.float32),
        mesh=mesh,
        scratch_types=[
            pltpu.VMEM((TOK_W,), jnp.int32),
            pltpu.VMEM((TOK_W,), jnp.int32),
            pltpu.VMEM((CH,), jnp.int32),
            pltpu.VMEM((CH,), jnp.int32),
            pltpu.VMEM((CH, DIM), jnp.float32),
            pltpu.SemaphoreType.DMA,
        ],
    )(x2, rows0, rows1)


def _gemm_kernel(te_ref, tv_ref, xs_ref, w1_ref, b1_ref, w2_ref, b2_ref, y_ref):
    i = pl.program_id(0)

    @pl.when(tv_ref[i] == 1)
    def _():
        xb = xs_ref[...].astype(jnp.bfloat16)
        h = jnp.dot(xb, w1_ref[0], preferred_element_type=jnp.float32) + b1_ref[0]
        h = 0.5 * h * (1.0 + lax.erf(h * _SQRT1_2))
        y_ref[...] = jnp.dot(h.astype(jnp.bfloat16), w2_ref[0],
                             preferred_element_type=jnp.float32) + b2_ref[0]


def _gemm(te, tv, xs, W1b, b1r, W2b, b2r, interpret=False):
    grid_spec = pltpu.PrefetchScalarGridSpec(
        num_scalar_prefetch=2,
        grid=(NT,),
        in_specs=[
            pl.BlockSpec((T, DIM), lambda i, te, tv: (i, 0)),
            pl.BlockSpec((1, DIM, HT), lambda i, te, tv: (te[i], 0, 0)),
            pl.BlockSpec((1, 1, HT), lambda i, te, tv: (te[i], 0, 0)),
            pl.BlockSpec((1, HT, DIM), lambda i, te, tv: (te[i], 0, 0)),
            pl.BlockSpec((1, 1, DIM), lambda i, te, tv: (te[i], 0, 0)),
        ],
        out_specs=pl.BlockSpec((T, DIM), lambda i, te, tv: (i, 0)),
    )
    return pl.pallas_call(
        _gemm_kernel,
        grid_spec=grid_spec,
        out_shape=jax.ShapeDtypeStruct((ROWS, DIM), jnp.float32),
        compiler_params=pltpu.CompilerParams(
            dimension_semantics=("arbitrary",)),
        interpret=interpret,
    )(te, tv, xs, W1b, b1r, W2b, b2r)


def _gather_body(y_hbm, rows0_hbm, rows1_hbm, g0_hbm, g1_hbm,
                 i0_v, i1_v, idx0_v, idx1_v, y0_v, y1_v, sem):
    c = lax.axis_index("c")
    s_ = lax.axis_index("s")
    wid = s_ * NC + c
    base = wid * TOK_W

    pltpu.sync_copy(rows0_hbm.at[pl.ds(base, TOK_W)], i0_v)
    pltpu.sync_copy(rows1_hbm.at[pl.ds(base, TOK_W)], i1_v)
    for ch in range(TOK_W // CH):
        for k in range(CH // 16):
            idx0_v[pl.ds(k * 16, 16)] = i0_v[pl.ds(ch * CH + k * 16, 16)]
            idx1_v[pl.ds(k * 16, 16)] = i1_v[pl.ds(ch * CH + k * 16, 16)]
        cp0 = pltpu.async_copy(y_hbm.at[idx0_v], y0_v, sem)
        cp1 = pltpu.async_copy(y_hbm.at[idx1_v], y1_v, sem)
        cp0.wait()
        cp1.wait()
        pltpu.sync_copy(y0_v, g0_hbm.at[pl.ds(base + ch * CH, CH)])
        pltpu.sync_copy(y1_v, g1_hbm.at[pl.ds(base + ch * CH, CH)])


def _gather(y, rows0, rows1, interpret=False):
    del interpret
    mesh = plsc.VectorSubcoreMesh(core_axis_name="c", subcore_axis_name="s")
    return pl.kernel(
        _gather_body,
        out_type=[
            jax.ShapeDtypeStruct((N, DIM), jnp.float32),
            jax.ShapeDtypeStruct((N, DIM), jnp.float32),
        ],
        mesh=mesh,
        scratch_types=[
            pltpu.VMEM((TOK_W,), jnp.int32),
            pltpu.VMEM((TOK_W,), jnp.int32),
            pltpu.VMEM((CH,), jnp.int32),
            pltpu.VMEM((CH,), jnp.int32),
            pltpu.VMEM((CH, DIM), jnp.float32),
            pltpu.VMEM((CH, DIM), jnp.float32),
            pltpu.SemaphoreType.DMA,
        ],
    )(y, rows0, rows1)


def _mix_kernel(g0_ref, g1_ref, wc0_ref, wc1_ref, out_ref):
    out_ref[...] = wc0_ref[...] * g0_ref[...] + wc1_ref[...] * g1_ref[...]


def _mix(g0, g1, wc0, wc1, interpret=False):
    return pl.pallas_call(
        _mix_kernel,
        grid=(N // TN2,),
        in_specs=[
            pl.BlockSpec((TN2, DIM), lambda t: (t, 0)),
            pl.BlockSpec((TN2, DIM), lambda t: (t, 0)),
            pl.BlockSpec((TN2, 1), lambda t: (t, 0)),
            pl.BlockSpec((TN2, 1), lambda t: (t, 0)),
        ],
        out_specs=pl.BlockSpec((TN2, DIM), lambda t: (t, 0)),
        out_shape=jax.ShapeDtypeStruct((N, DIM), jnp.float32),
        interpret=interpret,
    )(g0, g1, wc0, wc1)


def kernel(x, Wg, bg, W1, b1, W2, b2, interpret=False):
    x2 = x.reshape(N, DIM)
    Wgp = jnp.zeros((DIM, EP), jnp.float32).at[:, :E].set(Wg)
    bgp = jnp.full((EP, 1), -1e30, jnp.float32).at[:E, 0].set(bg)
    rows0, rows1, wc0, wc1, te, tv = _gate(x2, Wgp, bgp, interpret=interpret)
    r0f = rows0.reshape(N)
    r1f = rows1.reshape(N)
    xs = _scatter(x2, r0f, r1f, interpret=interpret)
    W1b = W1.astype(jnp.bfloat16)
    W2b = W2.astype(jnp.bfloat16)
    b1r = b1.reshape(E, 1, HID)
    b2r = b2.reshape(E, 1, DIM)
    y = _gemm(te.reshape(EP)[:NTP], tv.reshape(EP)[:NTP], xs, W1b, b1r, W2b,
              b2r, interpret=interpret)
    g0, g1 = _gather(y, r0f, r1f, interpret=interpret)
    out2 = _mix(g0, g1, wc0, wc1, interpret=interpret)
    return out2.reshape(B, S, DIM)
